# baseline (speedup 1.0000x reference)
import functools

import jax
import jax.numpy as jnp
from jax import lax
from jax.experimental import pallas as pl
from jax.experimental.pallas import tpu as pltpu
from jax.experimental.pallas import tpu_sc as plsc

N = 10000
NPAD = 10112
E = 160000
NC = 2
NS = 16
L = 16

CH = 104
NCHUNK = 97
EPT = NCHUNK * CH
EP = EPT * NS
RPT = NPAD // NS

EPTD = E // NC // NS
CHD = 125
NCHD = EPTD // CHD
CH2 = 100
NCH2 = EPTD // CH2

_MESH = plsc.VectorSubcoreMesh(core_axis_name="c", subcore_axis_name="s")


def _make_deg():
  @functools.partial(
      pl.kernel,
      out_type=jax.ShapeDtypeStruct((NC, NPAD, 128), jnp.float32),
      mesh=_MESH,
      scratch_types=[
          pltpu.VMEM_SHARED((NPAD, 128), jnp.float32),
          pltpu.VMEM((CHD, 128), jnp.float32),
          pltpu.VMEM((NCHD, CHD), jnp.int32),
          pltpu.SemaphoreType.DMA,
      ],
  )
  def deg_kernel(dst4, ones_hbm, zer_hbm, out, d_sh, ones_v, dst_v, sem):
    c = lax.axis_index("c")
    s = lax.axis_index("s")

    pltpu.sync_copy(zer_hbm, d_sh.at[pl.ds(s * RPT, RPT)])
    pltpu.sync_copy(ones_hbm, ones_v)
    pltpu.sync_copy(dst4.at[c, s], dst_v)
    plsc.subcore_barrier()

    def fire(j, carry):
      pltpu.async_copy(ones_v, d_sh.at[dst_v.at[j]], sem, add=True)
      return carry

    lax.fori_loop(0, NCHD, fire, 0)

    def drain(j, carry):
      pltpu.make_async_copy(ones_v, d_sh.at[dst_v.at[j]], sem).wait()
      return carry

    lax.fori_loop(0, NCHD, drain, 0)
    plsc.subcore_barrier()
    pltpu.sync_copy(d_sh.at[pl.ds(s * RPT, RPT)],
                    out.at[c, pl.ds(s * RPT, RPT)])

  return deg_kernel


def _pipelined_chunks(h, s_sh, src_row, dst_v, nchunk,
                      g0, g1, sg0, sg1, ss0, ss1):
  bufs = ((g0, sg0, ss0), (g1, sg1, ss1))

  pltpu.async_copy(h.at[src_row(0)], g0, sg0)

  def pair(p, carry):
    j0 = 2 * p
    for b in range(2):
      j = j0 + b
      g, sg, ss = bufs[b]
      go, sgo, sso = bufs[1 - b]

      @pl.when(j < nchunk)
      def _():
        pltpu.make_async_copy(h.at[src_row(j)], g, sg).wait()

        @pl.when(j + 1 < nchunk)
        def _():
          @pl.when(j >= 1)
          def _():
            pltpu.make_async_copy(go, s_sh.at[dst_v.at[j - 1]], sso).wait()

          pltpu.async_copy(h.at[src_row(j + 1)], go, sgo)

        pltpu.async_copy(g, s_sh.at[dst_v.at[j]], ss, add=True)
    return carry

  lax.fori_loop(0, (nchunk + 1) // 2, pair, 0)

  last = nchunk - 1
  gl, _, ssl = bufs[last % 2]
  pltpu.make_async_copy(gl, s_sh.at[dst_v.at[last]], ssl).wait()
  if nchunk >= 2:
    gp, _, ssp = bufs[(last - 1) % 2]
    pltpu.make_async_copy(gp, s_sh.at[dst_v.at[last - 1]], ssp).wait()


def _make_agg_feat():
  @functools.partial(
      pl.kernel,
      out_type=jax.ShapeDtypeStruct((NC, NPAD, 128), jnp.float32),
      mesh=_MESH,
      scratch_types=[
          pltpu.VMEM_SHARED((NPAD, 128), jnp.float32),
          pltpu.VMEM((EPT,), jnp.int32),
          pltpu.VMEM((NCHUNK, CH), jnp.int32),
          pltpu.VMEM((CH, 128), jnp.float32),
          pltpu.VMEM((CH, 128), jnp.float32),
          pltpu.SemaphoreType.DMA,
          pltpu.SemaphoreType.DMA,
          pltpu.SemaphoreType.DMA,
          pltpu.SemaphoreType.DMA,
      ],
  )
  def agg_kernel(h0, h1, src2, dst3, zer_hbm, out,
                 s_sh, src_v, dst_v, g0, g1, sg0, sg1, ss0, ss1):
    c = lax.axis_index("c")
    s = lax.axis_index("s")

    pltpu.sync_copy(zer_hbm, s_sh.at[pl.ds(s * RPT, RPT)])
    pltpu.sync_copy(src2.at[s], src_v)
    pltpu.sync_copy(dst3.at[s], dst_v)
    plsc.subcore_barrier()

    def src_row(j):
      return src_v.at[pl.ds(j * CH, CH)]

    @pl.when(c == 0)
    def _():
      _pipelined_chunks(h0, s_sh, src_row, dst_v, NCHUNK,
                        g0, g1, sg0, sg1, ss0, ss1)

    @pl.when(c == 1)
    def _():
      _pipelined_chunks(h1, s_sh, src_row, dst_v, NCHUNK,
                        g0, g1, sg0, sg1, ss0, ss1)

    plsc.subcore_barrier()
    pltpu.sync_copy(s_sh.at[pl.ds(s * RPT, RPT)],
                    out.at[c, pl.ds(s * RPT, RPT)])

  return agg_kernel


def _make_agg_edge():
  @functools.partial(
      pl.kernel,
      out_type=jax.ShapeDtypeStruct((NC, NPAD, 128), jnp.float32),
      mesh=_MESH,
      scratch_types=[
          pltpu.VMEM_SHARED((NPAD, 128), jnp.float32),
          pltpu.VMEM((NCH2, CH2), jnp.int32),
          pltpu.VMEM((NCH2, CH2), jnp.int32),
          pltpu.VMEM((CH2, 128), jnp.float32),
          pltpu.VMEM((CH2, 128), jnp.float32),
          pltpu.SemaphoreType.DMA,
          pltpu.SemaphoreType.DMA,
          pltpu.SemaphoreType.DMA,
          pltpu.SemaphoreType.DMA,
      ],
  )
  def agg_kernel(h, src4, dst4, zer_hbm, out,
                 s_sh, src_v, dst_v, g0, g1, sg0, sg1, ss0, ss1):
    c = lax.axis_index("c")
    s = lax.axis_index("s")

    pltpu.sync_copy(zer_hbm, s_sh.at[pl.ds(s * RPT, RPT)])
    pltpu.sync_copy(src4.at[c, s], src_v)
    pltpu.sync_copy(dst4.at[c, s], dst_v)
    plsc.subcore_barrier()

    def src_row(j):
      return src_v.at[j]

    _pipelined_chunks(h, s_sh, src_row, dst_v, NCH2,
                      g0, g1, sg0, sg1, ss0, ss1)
    plsc.subcore_barrier()
    pltpu.sync_copy(s_sh.at[pl.ds(s * RPT, RPT)],
                    out.at[c, pl.ds(s * RPT, RPT)])

  return agg_kernel


_deg = _make_deg()
_agg128 = _make_agg_feat()
_agg2 = _make_agg_edge()


_R = 1000


def _dinv_of(dg):
  return lax.rsqrt(dg[0, :, 0:1] + dg[1, :, 0:1] + 1.0)


def _tc1_body(x_ref, w_ref, dg_ref, o_ref):
  dinv = _dinv_of(dg_ref[...])
  p = jnp.dot(x_ref[...], w_ref[...], preferred_element_type=jnp.float32)
  o_ref[...] = (p * dinv)[None]


def _tc1(x, W1, degs):
  return pl.pallas_call(
      _tc1_body,
      grid=(NC, N // _R),
      in_specs=[
          pl.BlockSpec((_R, 256), lambda c, r: (r, 0)),
          pl.BlockSpec((256, 128), lambda c, r: (0, c)),
          pl.BlockSpec((2, _R, 128), lambda c, r: (0, r, 0)),
      ],
      out_specs=pl.BlockSpec((1, _R, 128), lambda c, r: (c, r, 0)),
      out_shape=jax.ShapeDtypeStruct((NC, N, 128), jnp.float32),
  )(x, W1, degs)


def _tc2_body(s1_ref, h1_ref, dg_ref, b1_ref, w2_ref, o_ref):
  dinv = _dinv_of(dg_ref[...])
  zl = jnp.maximum((s1_ref[0] + h1_ref[0]) * dinv + b1_ref[0:1, 0:128], 0.0)
  zh = jnp.maximum((s1_ref[1] + h1_ref[1]) * dinv + b1_ref[0:1, 128:256], 0.0)
  z = jnp.concatenate([zl, zh], axis=1)
  h2 = jnp.dot(z, w2_ref[...], preferred_element_type=jnp.float32)
  o_ref[...] = jnp.concatenate(
      [h2 * dinv, jnp.zeros((_R, 64), jnp.float32)], axis=1)


def _tc2(s1, h1s, degs, b1, W2):
  return pl.pallas_call(
      _tc2_body,
      grid=(N // _R,),
      in_specs=[
          pl.BlockSpec((2, _R, 128), lambda r: (0, r, 0)),
          pl.BlockSpec((2, _R, 128), lambda r: (0, r, 0)),
          pl.BlockSpec((2, _R, 128), lambda r: (0, r, 0)),
          pl.BlockSpec((1, 256), lambda r: (0, 0)),
          pl.BlockSpec((256, 64), lambda r: (0, 0)),
      ],
      out_specs=pl.BlockSpec((_R, 128), lambda r: (r, 0)),
      out_shape=jax.ShapeDtypeStruct((N, 128), jnp.float32),
  )(s1, h1s, degs, b1, W2)


def _tc3_body(s2_ref, h2_ref, dg_ref, b2_ref, o_ref):
  dinv = _dinv_of(dg_ref[...])
  f = (s2_ref[0] + s2_ref[1] + h2_ref[...])[:, 0:64]
  v = f * dinv + b2_ref[0:1, :]
  m = jnp.max(v, axis=1, keepdims=True)
  lse = jnp.log(jnp.sum(jnp.exp(v - m), axis=1, keepdims=True)) + m
  o_ref[...] = v - lse


def _tc3(s2, h2p, degs, b2):
  return pl.pallas_call(
      _tc3_body,
      grid=(N // _R,),
      in_specs=[
          pl.BlockSpec((2, _R, 128), lambda r: (0, r, 0)),
          pl.BlockSpec((_R, 128), lambda r: (r, 0)),
          pl.BlockSpec((2, _R, 128), lambda r: (0, r, 0)),
          pl.BlockSpec((1, 64), lambda r: (0, 0)),
      ],
      out_specs=pl.BlockSpec((_R, 64), lambda r: (r, 0)),
      out_shape=jax.ShapeDtypeStruct((N, 64), jnp.float32),
  )(s2, h2p, degs, b2)


def kernel(x, edge_index, W1, b1, W2, b2):
  src = edge_index[0].astype(jnp.int32)
  dst = edge_index[1].astype(jnp.int32)
  npadE = EP - E
  srcP = jnp.concatenate([src, jnp.zeros((npadE,), jnp.int32)])
  dstP = jnp.concatenate([dst, jnp.full((npadE,), NPAD - 1, jnp.int32)])
  src2 = srcP.reshape(NS, EPT)
  dst3 = dstP.reshape(NS, NCHUNK, CH)
  srcD = src.reshape(NC, NS, NCH2, CH2)
  dstD = dst.reshape(NC, NS, NCH2, CH2)
  dstH = dst.reshape(NC, NS, NCHD, CHD)

  ones128 = jnp.ones((CHD, 128), jnp.float32)
  zer128 = jnp.zeros((RPT, 128), jnp.float32)

  degs = _deg(dstH, ones128, zer128)
  h1s = _tc1(x, W1, degs)
  s1 = _agg128(h1s[0], h1s[1], src2, dst3, zer128)
  h2p = _tc2(s1, h1s, degs, b1.reshape(1, 256), W2)
  s2 = _agg2(h2p, srcD, dstD, zer128)
  return _tc3(s2, h2p, degs, b2.reshape(1, 64))


# speedup vs baseline: 1.1473x; 1.1473x over previous
import functools

import jax
import jax.numpy as jnp
from jax import lax
from jax.experimental import pallas as pl
from jax.experimental.pallas import tpu as pltpu
from jax.experimental.pallas import tpu_sc as plsc

N = 10000
NPAD = 10240
E = 160000
NC = 2
NS = 16
L = 16

EPT = E // NS
CH = 80
NCHUNK = EPT // CH
RPT = NPAD // NS

EPTD = E // NC // NS
CHD = 125
NCHD = EPTD // CHD
CH2 = 100
NCH2 = EPTD // CH2

_MESH = plsc.VectorSubcoreMesh(core_axis_name="c", subcore_axis_name="s")


def _make_deg():
  @functools.partial(
      pl.kernel,
      out_type=jax.ShapeDtypeStruct((NC, NPAD, 128), jnp.float32),
      mesh=_MESH,
      scratch_types=[
          pltpu.VMEM_SHARED((NPAD, 128), jnp.float32),
          pltpu.VMEM((CHD, 128), jnp.float32),
          pltpu.VMEM((NCHD, CHD), jnp.int32),
          pltpu.SemaphoreType.DMA,
      ],
  )
  def deg_kernel(dst4, ones_hbm, zer_hbm, out, d_sh, ones_v, dst_v, sem):
    c = lax.axis_index("c")
    s = lax.axis_index("s")

    pltpu.sync_copy(zer_hbm, d_sh.at[pl.ds(s * RPT, RPT)])
    pltpu.sync_copy(ones_hbm, ones_v)
    pltpu.sync_copy(dst4.at[c, s], dst_v)
    plsc.subcore_barrier()

    def fire(j, carry):
      pltpu.async_copy(ones_v, d_sh.at[dst_v.at[j]], sem, add=True)
      return carry

    lax.fori_loop(0, NCHD, fire, 0)

    def drain(j, carry):
      pltpu.make_async_copy(ones_v, d_sh.at[dst_v.at[j]], sem).wait()
      return carry

    lax.fori_loop(0, NCHD, drain, 0)
    plsc.subcore_barrier()
    pltpu.sync_copy(d_sh.at[pl.ds(s * RPT, RPT)],
                    out.at[c, pl.ds(s * RPT, RPT)])

  return deg_kernel


def _pipelined_chunks(h, s_sh, src_row, dst_v, nchunk,
                      g0, g1, sg0, sg1, ss0, ss1):
  bufs = ((g0, sg0, ss0), (g1, sg1, ss1))

  pltpu.async_copy(h.at[src_row(0)], g0, sg0)

  def pair(p, carry):
    j0 = 2 * p
    for b in range(2):
      j = j0 + b
      g, sg, ss = bufs[b]
      go, sgo, sso = bufs[1 - b]

      @pl.when(j < nchunk)
      def _():
        pltpu.make_async_copy(h.at[src_row(j)], g, sg).wait()

        @pl.when(j + 1 < nchunk)
        def _():
          @pl.when(j >= 1)
          def _():
            pltpu.make_async_copy(go, s_sh.at[dst_v.at[j - 1]], sso).wait()

          pltpu.async_copy(h.at[src_row(j + 1)], go, sgo)

        pltpu.async_copy(g, s_sh.at[dst_v.at[j]], ss, add=True)
    return carry

  lax.fori_loop(0, (nchunk + 1) // 2, pair, 0)

  last = nchunk - 1
  gl, _, ssl = bufs[last % 2]
  pltpu.make_async_copy(gl, s_sh.at[dst_v.at[last]], ssl).wait()
  if nchunk >= 2:
    gp, _, ssp = bufs[(last - 1) % 2]
    pltpu.make_async_copy(gp, s_sh.at[dst_v.at[last - 1]], ssp).wait()


def _make_agg_feat():
  @functools.partial(
      pl.kernel,
      out_type=jax.ShapeDtypeStruct((NC, NPAD, 128), jnp.float32),
      mesh=_MESH,
      scratch_types=[
          pltpu.VMEM_SHARED((NPAD, 128), jnp.float32),
          pltpu.VMEM((EPT,), jnp.int32),
          pltpu.VMEM((NCHUNK, CH), jnp.int32),
          pltpu.VMEM((CH, 128), jnp.float32),
          pltpu.VMEM((CH, 128), jnp.float32),
          pltpu.SemaphoreType.DMA,
          pltpu.SemaphoreType.DMA,
          pltpu.SemaphoreType.DMA,
          pltpu.SemaphoreType.DMA,
      ],
  )
  def agg_kernel(h0, h1, src2, dst3, zer_hbm, out,
                 s_sh, src_v, dst_v, g0, g1, sg0, sg1, ss0, ss1):
    c = lax.axis_index("c")
    s = lax.axis_index("s")

    pltpu.sync_copy(zer_hbm, s_sh.at[pl.ds(s * RPT, RPT)])
    pltpu.sync_copy(src2.at[s], src_v)
    pltpu.sync_copy(dst3.at[s], dst_v)
    plsc.subcore_barrier()

    def src_row(j):
      return src_v.at[pl.ds(j * CH, CH)]

    @pl.when(c == 0)
    def _():
      _pipelined_chunks(h0, s_sh, src_row, dst_v, NCHUNK,
                        g0, g1, sg0, sg1, ss0, ss1)

    @pl.when(c == 1)
    def _():
      _pipelined_chunks(h1, s_sh, src_row, dst_v, NCHUNK,
                        g0, g1, sg0, sg1, ss0, ss1)

    plsc.subcore_barrier()
    pltpu.sync_copy(s_sh.at[pl.ds(s * RPT, RPT)],
                    out.at[c, pl.ds(s * RPT, RPT)])

  return agg_kernel


def _make_agg_edge():
  @functools.partial(
      pl.kernel,
      out_type=jax.ShapeDtypeStruct((NC, NPAD, 128), jnp.float32),
      mesh=_MESH,
      scratch_types=[
          pltpu.VMEM_SHARED((NPAD, 128), jnp.float32),
          pltpu.VMEM((NCH2, CH2), jnp.int32),
          pltpu.VMEM((NCH2, CH2), jnp.int32),
          pltpu.VMEM((CH2, 128), jnp.float32),
          pltpu.VMEM((CH2, 128), jnp.float32),
          pltpu.SemaphoreType.DMA,
          pltpu.SemaphoreType.DMA,
          pltpu.SemaphoreType.DMA,
          pltpu.SemaphoreType.DMA,
      ],
  )
  def agg_kernel(h, src4, dst4, zer_hbm, out,
                 s_sh, src_v, dst_v, g0, g1, sg0, sg1, ss0, ss1):
    c = lax.axis_index("c")
    s = lax.axis_index("s")

    pltpu.sync_copy(zer_hbm, s_sh.at[pl.ds(s * RPT, RPT)])
    pltpu.sync_copy(src4.at[c, s], src_v)
    pltpu.sync_copy(dst4.at[c, s], dst_v)
    plsc.subcore_barrier()

    def src_row(j):
      return src_v.at[j]

    _pipelined_chunks(h, s_sh, src_row, dst_v, NCH2,
                      g0, g1, sg0, sg1, ss0, ss1)
    plsc.subcore_barrier()
    pltpu.sync_copy(s_sh.at[pl.ds(s * RPT, RPT)],
                    out.at[c, pl.ds(s * RPT, RPT)])

  return agg_kernel


_deg = _make_deg()
_agg128 = _make_agg_feat()
_agg2 = _make_agg_edge()


_R = 1000


def _dinv_of(dg):
  return lax.rsqrt(dg[0, :, 0:1] + dg[1, :, 0:1] + 1.0)


def _tc1_body(x_ref, w_ref, dg_ref, o_ref):
  dinv = _dinv_of(dg_ref[...])
  p = jnp.dot(x_ref[...], w_ref[...], preferred_element_type=jnp.float32)
  o_ref[...] = (p * dinv)[None]


def _tc1(x, W1, degs):
  return pl.pallas_call(
      _tc1_body,
      grid=(NC, N // _R),
      in_specs=[
          pl.BlockSpec((_R, 256), lambda c, r: (r, 0)),
          pl.BlockSpec((256, 128), lambda c, r: (0, c)),
          pl.BlockSpec((2, _R, 128), lambda c, r: (0, r, 0)),
      ],
      out_specs=pl.BlockSpec((1, _R, 128), lambda c, r: (c, r, 0)),
      out_shape=jax.ShapeDtypeStruct((NC, N, 128), jnp.float32),
  )(x, W1, degs)


def _tc2_body(s1_ref, h1_ref, dg_ref, b1_ref, w2_ref, o_ref):
  dinv = _dinv_of(dg_ref[...])
  zl = jnp.maximum((s1_ref[0] + h1_ref[0]) * dinv + b1_ref[0:1, 0:128], 0.0)
  zh = jnp.maximum((s1_ref[1] + h1_ref[1]) * dinv + b1_ref[0:1, 128:256], 0.0)
  z = jnp.concatenate([zl, zh], axis=1)
  h2 = jnp.dot(z, w2_ref[...], preferred_element_type=jnp.float32)
  o_ref[...] = jnp.concatenate(
      [h2 * dinv, jnp.zeros((_R, 64), jnp.float32)], axis=1)


def _tc2(s1, h1s, degs, b1, W2):
  return pl.pallas_call(
      _tc2_body,
      grid=(N // _R,),
      in_specs=[
          pl.BlockSpec((2, _R, 128), lambda r: (0, r, 0)),
          pl.BlockSpec((2, _R, 128), lambda r: (0, r, 0)),
          pl.BlockSpec((2, _R, 128), lambda r: (0, r, 0)),
          pl.BlockSpec((1, 256), lambda r: (0, 0)),
          pl.BlockSpec((256, 64), lambda r: (0, 0)),
      ],
      out_specs=pl.BlockSpec((_R, 128), lambda r: (r, 0)),
      out_shape=jax.ShapeDtypeStruct((N, 128), jnp.float32),
  )(s1, h1s, degs, b1, W2)


def _tc3_body(s2_ref, h2_ref, dg_ref, b2_ref, o_ref):
  dinv = _dinv_of(dg_ref[...])
  f = (s2_ref[0] + s2_ref[1] + h2_ref[...])[:, 0:64]
  v = f * dinv + b2_ref[0:1, :]
  m = jnp.max(v, axis=1, keepdims=True)
  lse = jnp.log(jnp.sum(jnp.exp(v - m), axis=1, keepdims=True)) + m
  o_ref[...] = v - lse


def _tc3(s2, h2p, degs, b2):
  return pl.pallas_call(
      _tc3_body,
      grid=(N // _R,),
      in_specs=[
          pl.BlockSpec((2, _R, 128), lambda r: (0, r, 0)),
          pl.BlockSpec((_R, 128), lambda r: (r, 0)),
          pl.BlockSpec((2, _R, 128), lambda r: (0, r, 0)),
          pl.BlockSpec((1, 64), lambda r: (0, 0)),
      ],
      out_specs=pl.BlockSpec((_R, 64), lambda r: (r, 0)),
      out_shape=jax.ShapeDtypeStruct((N, 64), jnp.float32),
  )(s2, h2p, degs, b2)


def kernel(x, edge_index, W1, b1, W2, b2):
  src = edge_index[0].astype(jnp.int32)
  dst = edge_index[1].astype(jnp.int32)
  src2 = src.reshape(NS, EPT)
  dst3 = dst.reshape(NS, NCHUNK, CH)
  srcD = src.reshape(NC, NS, NCH2, CH2)
  dstD = dst.reshape(NC, NS, NCH2, CH2)
  dstH = dst.reshape(NC, NS, NCHD, CHD)

  ones128 = jnp.ones((CHD, 128), jnp.float32)
  zer128 = jnp.zeros((RPT, 128), jnp.float32)

  degs = _deg(dstH, ones128, zer128)
  h1s = _tc1(x, W1, degs)
  s1 = _agg128(h1s[0], h1s[1], src2, dst3, zer128)
  h2p = _tc2(s1, h1s, degs, b1.reshape(1, 256), W2)
  s2 = _agg2(h2p, srcD, dstD, zer128)
  return _tc3(s2, h2p, degs, b2.reshape(1, 64))


# speedup vs baseline: 1.1645x; 1.0150x over previous
import functools

import jax
import jax.numpy as jnp
from jax import lax
from jax.experimental import pallas as pl
from jax.experimental.pallas import tpu as pltpu
from jax.experimental.pallas import tpu_sc as plsc

N = 10000
NPAD = 10240
E = 160000
NC = 2
NS = 16
L = 16

EPT = E // NS
CH = 80
NCHUNK = EPT // CH
RPT = NPAD // NS

EPTD = E // NC // NS
CHD = 125
NCHD = EPTD // CHD
CH2 = 125
NCH2 = EPTD // CH2

_MESH = plsc.VectorSubcoreMesh(core_axis_name="c", subcore_axis_name="s")


def _make_deg():
  @functools.partial(
      pl.kernel,
      out_type=jax.ShapeDtypeStruct((NC, NPAD, 128), jnp.float32),
      mesh=_MESH,
      scratch_types=[
          pltpu.VMEM_SHARED((NPAD, 128), jnp.float32),
          pltpu.VMEM((CHD, 128), jnp.float32),
          pltpu.VMEM((NCHD, CHD), jnp.int32),
          pltpu.SemaphoreType.DMA,
      ],
  )
  def deg_kernel(dst4, ones_hbm, zer_hbm, out, d_sh, ones_v, dst_v, sem):
    c = lax.axis_index("c")
    s = lax.axis_index("s")

    pltpu.sync_copy(zer_hbm, d_sh.at[pl.ds(s * RPT, RPT)])
    pltpu.sync_copy(ones_hbm, ones_v)
    pltpu.sync_copy(dst4.at[c, s], dst_v)
    plsc.subcore_barrier()

    def fire(j, carry):
      pltpu.async_copy(ones_v, d_sh.at[dst_v.at[j]], sem, add=True)
      return carry

    lax.fori_loop(0, NCHD, fire, 0)

    def drain(j, carry):
      pltpu.make_async_copy(ones_v, d_sh.at[dst_v.at[j]], sem).wait()
      return carry

    lax.fori_loop(0, NCHD, drain, 0)
    plsc.subcore_barrier()
    pltpu.sync_copy(d_sh.at[pl.ds(s * RPT, RPT)],
                    out.at[c, pl.ds(s * RPT, RPT)])

  return deg_kernel


def _pipelined_chunks(h, s_sh, src_row, dst_v, nchunk,
                      g0, g1, sg0, sg1, ss0, ss1):
  bufs = ((g0, sg0, ss0), (g1, sg1, ss1))

  pltpu.async_copy(h.at[src_row(0)], g0, sg0)

  def pair(p, carry):
    j0 = 2 * p
    for b in range(2):
      j = j0 + b
      g, sg, ss = bufs[b]
      go, sgo, sso = bufs[1 - b]

      @pl.when(j < nchunk)
      def _():
        pltpu.make_async_copy(h.at[src_row(j)], g, sg).wait()

        @pl.when(j + 1 < nchunk)
        def _():
          @pl.when(j >= 1)
          def _():
            pltpu.make_async_copy(go, s_sh.at[dst_v.at[j - 1]], sso).wait()

          pltpu.async_copy(h.at[src_row(j + 1)], go, sgo)

        pltpu.async_copy(g, s_sh.at[dst_v.at[j]], ss, add=True)
    return carry

  lax.fori_loop(0, (nchunk + 1) // 2, pair, 0)

  last = nchunk - 1
  gl, _, ssl = bufs[last % 2]
  pltpu.make_async_copy(gl, s_sh.at[dst_v.at[last]], ssl).wait()
  if nchunk >= 2:
    gp, _, ssp = bufs[(last - 1) % 2]
    pltpu.make_async_copy(gp, s_sh.at[dst_v.at[last - 1]], ssp).wait()


def _make_agg_feat():
  @functools.partial(
      pl.kernel,
      out_type=jax.ShapeDtypeStruct((NC, NPAD, 128), jnp.float32),
      mesh=_MESH,
      scratch_types=[
          pltpu.VMEM_SHARED((NPAD, 128), jnp.float32),
          pltpu.VMEM((EPT,), jnp.int32),
          pltpu.VMEM((NCHUNK, CH), jnp.int32),
          pltpu.VMEM((CH, 128), jnp.float32),
          pltpu.VMEM((CH, 128), jnp.float32),
          pltpu.SemaphoreType.DMA,
          pltpu.SemaphoreType.DMA,
          pltpu.SemaphoreType.DMA,
          pltpu.SemaphoreType.DMA,
      ],
  )
  def agg_kernel(h0, h1, src2, dst3, zer_hbm, out,
                 s_sh, src_v, dst_v, g0, g1, sg0, sg1, ss0, ss1):
    c = lax.axis_index("c")
    s = lax.axis_index("s")

    pltpu.sync_copy(zer_hbm, s_sh.at[pl.ds(s * RPT, RPT)])
    pltpu.sync_copy(src2.at[s], src_v)
    pltpu.sync_copy(dst3.at[s], dst_v)
    plsc.subcore_barrier()

    def src_row(j):
      return src_v.at[pl.ds(j * CH, CH)]

    @pl.when(c == 0)
    def _():
      _pipelined_chunks(h0, s_sh, src_row, dst_v, NCHUNK,
                        g0, g1, sg0, sg1, ss0, ss1)

    @pl.when(c == 1)
    def _():
      _pipelined_chunks(h1, s_sh, src_row, dst_v, NCHUNK,
                        g0, g1, sg0, sg1, ss0, ss1)

    plsc.subcore_barrier()
    pltpu.sync_copy(s_sh.at[pl.ds(s * RPT, RPT)],
                    out.at[c, pl.ds(s * RPT, RPT)])

  return agg_kernel


def _make_agg_edge():
  @functools.partial(
      pl.kernel,
      out_type=jax.ShapeDtypeStruct((NC, NPAD, 128), jnp.float32),
      mesh=_MESH,
      scratch_types=[
          pltpu.VMEM_SHARED((NPAD, 128), jnp.float32),
          pltpu.VMEM((NCH2, CH2), jnp.int32),
          pltpu.VMEM((NCH2, CH2), jnp.int32),
          pltpu.VMEM((CH2, 128), jnp.float32),
          pltpu.VMEM((CH2, 128), jnp.float32),
          pltpu.SemaphoreType.DMA,
          pltpu.SemaphoreType.DMA,
          pltpu.SemaphoreType.DMA,
          pltpu.SemaphoreType.DMA,
      ],
  )
  def agg_kernel(h, src4, dst4, zer_hbm, out,
                 s_sh, src_v, dst_v, g0, g1, sg0, sg1, ss0, ss1):
    c = lax.axis_index("c")
    s = lax.axis_index("s")

    pltpu.sync_copy(zer_hbm, s_sh.at[pl.ds(s * RPT, RPT)])
    pltpu.sync_copy(src4.at[c, s], src_v)
    pltpu.sync_copy(dst4.at[c, s], dst_v)
    plsc.subcore_barrier()

    def src_row(j):
      return src_v.at[j]

    _pipelined_chunks(h, s_sh, src_row, dst_v, NCH2,
                      g0, g1, sg0, sg1, ss0, ss1)
    plsc.subcore_barrier()
    pltpu.sync_copy(s_sh.at[pl.ds(s * RPT, RPT)],
                    out.at[c, pl.ds(s * RPT, RPT)])

  return agg_kernel


_deg = _make_deg()
_agg128 = _make_agg_feat()
_agg2 = _make_agg_edge()


_R = 1000


def _dinv_of(dg):
  return lax.rsqrt(dg[0, :, 0:1] + dg[1, :, 0:1] + 1.0)


def _tc1_body(x_ref, w_ref, dg_ref, o_ref):
  dinv = _dinv_of(dg_ref[...])
  p = jnp.dot(x_ref[...], w_ref[...], preferred_element_type=jnp.float32)
  o_ref[...] = (p * dinv)[None]


def _tc1(x, W1, degs):
  return pl.pallas_call(
      _tc1_body,
      grid=(NC, N // _R),
      in_specs=[
          pl.BlockSpec((_R, 256), lambda c, r: (r, 0)),
          pl.BlockSpec((256, 128), lambda c, r: (0, c)),
          pl.BlockSpec((2, _R, 128), lambda c, r: (0, r, 0)),
      ],
      out_specs=pl.BlockSpec((1, _R, 128), lambda c, r: (c, r, 0)),
      out_shape=jax.ShapeDtypeStruct((NC, N, 128), jnp.float32),
  )(x, W1, degs)


def _tc2_body(s1_ref, h1_ref, dg_ref, b1_ref, w2_ref, o_ref):
  dinv = _dinv_of(dg_ref[...])
  zl = jnp.maximum((s1_ref[0] + h1_ref[0]) * dinv + b1_ref[0:1, 0:128], 0.0)
  zh = jnp.maximum((s1_ref[1] + h1_ref[1]) * dinv + b1_ref[0:1, 128:256], 0.0)
  z = jnp.concatenate([zl, zh], axis=1)
  h2 = jnp.dot(z, w2_ref[...], preferred_element_type=jnp.float32)
  o_ref[...] = jnp.concatenate(
      [h2 * dinv, jnp.zeros((_R, 64), jnp.float32)], axis=1)


def _tc2(s1, h1s, degs, b1, W2):
  return pl.pallas_call(
      _tc2_body,
      grid=(N // _R,),
      in_specs=[
          pl.BlockSpec((2, _R, 128), lambda r: (0, r, 0)),
          pl.BlockSpec((2, _R, 128), lambda r: (0, r, 0)),
          pl.BlockSpec((2, _R, 128), lambda r: (0, r, 0)),
          pl.BlockSpec((1, 256), lambda r: (0, 0)),
          pl.BlockSpec((256, 64), lambda r: (0, 0)),
      ],
      out_specs=pl.BlockSpec((_R, 128), lambda r: (r, 0)),
      out_shape=jax.ShapeDtypeStruct((N, 128), jnp.float32),
  )(s1, h1s, degs, b1, W2)


def _tc3_body(s2_ref, h2_ref, dg_ref, b2_ref, o_ref):
  dinv = _dinv_of(dg_ref[...])
  f = (s2_ref[0] + s2_ref[1] + h2_ref[...])[:, 0:64]
  v = f * dinv + b2_ref[0:1, :]
  m = jnp.max(v, axis=1, keepdims=True)
  lse = jnp.log(jnp.sum(jnp.exp(v - m), axis=1, keepdims=True)) + m
  o_ref[...] = v - lse


def _tc3(s2, h2p, degs, b2):
  return pl.pallas_call(
      _tc3_body,
      grid=(N // _R,),
      in_specs=[
          pl.BlockSpec((2, _R, 128), lambda r: (0, r, 0)),
          pl.BlockSpec((_R, 128), lambda r: (r, 0)),
          pl.BlockSpec((2, _R, 128), lambda r: (0, r, 0)),
          pl.BlockSpec((1, 64), lambda r: (0, 0)),
      ],
      out_specs=pl.BlockSpec((_R, 64), lambda r: (r, 0)),
      out_shape=jax.ShapeDtypeStruct((N, 64), jnp.float32),
  )(s2, h2p, degs, b2)


def kernel(x, edge_index, W1, b1, W2, b2):
  src = edge_index[0].astype(jnp.int32)
  dst = edge_index[1].astype(jnp.int32)
  src2 = src.reshape(NS, EPT)
  dst3 = dst.reshape(NS, NCHUNK, CH)
  srcD = src.reshape(NC, NS, NCH2, CH2)
  dstD = dst.reshape(NC, NS, NCH2, CH2)
  dstH = dst.reshape(NC, NS, NCHD, CHD)

  ones128 = jnp.ones((CHD, 128), jnp.float32)
  zer128 = jnp.zeros((RPT, 128), jnp.float32)

  degs = _deg(dstH, ones128, zer128)
  h1s = _tc1(x, W1, degs)
  s1 = _agg128(h1s[0], h1s[1], src2, dst3, zer128)
  h2p = _tc2(s1, h1s, degs, b1.reshape(1, 256), W2)
  s2 = _agg2(h2p, srcD, dstD, zer128)
  return _tc3(s2, h2p, degs, b2.reshape(1, 64))
